# baseline (speedup 1.0000x reference)
import jax
import jax.numpy as jnp
from jax import lax
from jax.experimental import pallas as pl
from jax.experimental.pallas import tpu as pltpu
from jax.experimental.pallas import tpu_sc as plsc

N_NODES = 50000
FEAT = 128
NBR = 7
BLK = 16
ROWS = BLK * NBR
NBLK = N_NODES // BLK
NW = 32
NBUF = 3
BPW = -(-NBLK // (NW * NBUF)) * NBUF


def _body(x_hbm, no_hbm, out_hbm,
          idx0, idx1, idx2, rows0, rows1, rows2, ob0, ob1, ob2,
          isem0, isem1, isem2, gsem0, gsem1, gsem2, osem0, osem1, osem2):
    idx = [idx0, idx1, idx2]
    rows = [rows0, rows1, rows2]
    ob = [ob0, ob1, ob2]
    isem = [isem0, isem1, isem2]
    gsem = [gsem0, gsem1, gsem2]
    osem = [osem0, osem1, osem2]

    cid = lax.axis_index("c")
    sid = lax.axis_index("s")
    wid = sid * 2 + cid
    start = wid * BPW
    cnt = jnp.minimum(NBLK - start, BPW)

    iota = lax.iota(jnp.int32, 16)
    seven_iota = iota * 7

    def idx_start(c, b):
        pltpu.async_copy(
            no_hbm.at[pl.ds((start + c) * ROWS, ROWS)], idx[b], isem[b])

    def idx_wait(c, b):
        pltpu.make_async_copy(
            no_hbm.at[pl.ds((start + c) * ROWS, ROWS)], idx[b],
            isem[b]).wait()

    def gather_start(b):
        pltpu.async_copy(x_hbm.at[idx[b]], rows[b], gsem[b])

    def gather_wait(b):
        pltpu.make_async_copy(x_hbm.at[idx[b]], rows[b], gsem[b]).wait()

    def out_start(c, b):
        pltpu.async_copy(
            ob[b], out_hbm.at[pl.ds((start + c) * BLK, BLK)], osem[b])

    def out_drain(b):
        pltpu.make_async_copy(
            ob[b], out_hbm.at[pl.ds(0, BLK)], osem[b]).wait()

    def compute(rows_v, out_v):
        for t in range(8):
            j0s = [seven_iota + (112 * t + k) for k in range(NBR)]
            row0s = [lax.shift_right_logical(j0, 7) for j0 in j0s]
            col0s = [lax.bitwise_and(j0, 127) for j0 in j0s]

            @plsc.parallel_loop(0, BLK, unroll=12)
            def _node_loop(n, t=t, row0s=row0s, col0s=col0s,
                           rows_v=rows_v, out_v=out_v):
                n7 = n * 7
                g = [plsc.load_gather(rows_v, [row0s[k] + n7, col0s[k]])
                     for k in range(NBR)]
                s = ((g[0] + g[1]) + (g[2] + g[3])) + ((g[4] + g[5]) + g[6])
                out_v[n, pl.ds(16 * t, 16)] = s * (1.0 / 7.0)

    idx_start(0, 0)
    idx_start(1, 1)
    idx_start(2, 2)
    idx_wait(0, 0)
    gather_start(0)
    idx_wait(1, 1)
    gather_start(1)

    @pl.loop(0, BPW, step=NBUF)
    def _block_loop(i):
        for b in range(NBUF):
            c = i + b
            b2 = (b + 2) % NBUF

            @pl.when(c < cnt)
            def _(c=c, b=b, b2=b2):
                gather_wait(b)

                @pl.when(c + NBUF < cnt)
                def _():
                    idx_start(c + NBUF, b)

                @pl.when(c + 2 < cnt)
                def _():
                    idx_wait(c + 2, b2)
                    gather_start(b2)

                @pl.when(c >= NBUF)
                def _():
                    out_drain(b)

                compute(rows[b], ob[b])
                out_start(c, b)

    out_drain(0)
    out_drain(1)
    out_drain(2)


def kernel(x, neigh_orders):
    mesh = plsc.VectorSubcoreMesh(core_axis_name="c", subcore_axis_name="s")
    f = pl.kernel(
        _body,
        out_type=jax.ShapeDtypeStruct((N_NODES, FEAT), jnp.float32),
        mesh=mesh,
        scratch_types=(
            [pltpu.VMEM((ROWS,), jnp.int32) for _ in range(NBUF)]
            + [pltpu.VMEM((ROWS, FEAT), jnp.float32) for _ in range(NBUF)]
            + [pltpu.VMEM((BLK, FEAT), jnp.float32) for _ in range(NBUF)]
            + [pltpu.SemaphoreType.DMA for _ in range(3 * NBUF)]
        ),
        compiler_params=pltpu.CompilerParams(needs_layout_passes=False),
    )
    return f(x, neigh_orders)


# speedup vs baseline: 1.5031x; 1.5031x over previous
import jax
import jax.numpy as jnp
from jax import lax
from jax.experimental import pallas as pl
from jax.experimental.pallas import tpu as pltpu
from jax.experimental.pallas import tpu_sc as plsc

N_NODES = 50000
FEAT = 128
NBR = 7
BLK = 16
ROWS = BLK * NBR
NBLK = N_NODES // BLK
NW = 32
NBUF = 3
BPW = -(-NBLK // (NW * NBUF)) * NBUF


def _body(x_hbm, no_hbm, out_hbm,
          idx0, idx1, idx2, rows0, rows1, rows2, ob0, ob1, ob2,
          isem0, isem1, isem2, gsem0, gsem1, gsem2, osem0, osem1, osem2):
    idx = [idx0, idx1, idx2]
    rows = [rows0, rows1, rows2]
    ob = [ob0, ob1, ob2]
    isem = [isem0, isem1, isem2]
    gsem = [gsem0, gsem1, gsem2]
    osem = [osem0, osem1, osem2]

    cid = lax.axis_index("c")
    sid = lax.axis_index("s")
    wid = sid * 2 + cid
    start = wid * BPW
    cnt = jnp.minimum(NBLK - start, BPW)

    iota = lax.iota(jnp.int32, 16)
    seven_iota = iota * 7

    def idx_start(c, b):
        pltpu.async_copy(
            no_hbm.at[pl.ds((start + c) * ROWS, ROWS)], idx[b], isem[b])

    def idx_wait(c, b):
        pltpu.make_async_copy(
            no_hbm.at[pl.ds((start + c) * ROWS, ROWS)], idx[b],
            isem[b]).wait()

    def gather_start(b):
        pltpu.async_copy(x_hbm.at[idx[b]], rows[b], gsem[b])

    def gather_wait(b):
        pltpu.make_async_copy(x_hbm.at[idx[b]], rows[b], gsem[b]).wait()

    def out_start(c, b):
        pltpu.async_copy(
            ob[b], out_hbm.at[pl.ds((start + c) * BLK, BLK)], osem[b])

    def out_drain(b):
        pltpu.make_async_copy(
            ob[b], out_hbm.at[pl.ds(0, BLK)], osem[b]).wait()

    def compute(rows_v, out_v):
        for t in range(8):
            j0s = [seven_iota + (112 * t + k) for k in range(NBR)]
            row0s = [lax.shift_right_logical(j0, 7) for j0 in j0s]
            col0s = [lax.bitwise_and(j0, 127) for j0 in j0s]

            @plsc.parallel_loop(0, BLK, unroll=8)
            def _node_loop(n, t=t, row0s=row0s, col0s=col0s,
                           rows_v=rows_v, out_v=out_v):
                n7 = n * 7
                g = [plsc.load_gather(rows_v, [row0s[k] + n7, col0s[k]])
                     for k in range(NBR)]
                s = ((g[0] + g[1]) + (g[2] + g[3])) + ((g[4] + g[5]) + g[6])
                out_v[n, pl.ds(16 * t, 16)] = s * (1.0 / 7.0)

    idx_start(0, 0)
    idx_start(1, 1)
    idx_start(2, 2)
    idx_wait(0, 0)
    gather_start(0)
    idx_wait(1, 1)
    gather_start(1)

    @pl.loop(0, BPW, step=NBUF)
    def _block_loop(i):
        for b in range(NBUF):
            c = i + b
            b2 = (b + 2) % NBUF

            @pl.when(c < cnt)
            def _(c=c, b=b, b2=b2):
                gather_wait(b)

                @pl.when(c + NBUF < cnt)
                def _():
                    idx_start(c + NBUF, b)

                @pl.when(c + 2 < cnt)
                def _():
                    idx_wait(c + 2, b2)
                    gather_start(b2)

                @pl.when(c >= NBUF)
                def _():
                    out_drain(b)

                compute(rows[b], ob[b])
                out_start(c, b)

    out_drain(0)
    out_drain(1)
    out_drain(2)


def kernel(x, neigh_orders):
    mesh = plsc.VectorSubcoreMesh(core_axis_name="c", subcore_axis_name="s")
    f = pl.kernel(
        _body,
        out_type=jax.ShapeDtypeStruct((N_NODES, FEAT), jnp.float32),
        mesh=mesh,
        scratch_types=(
            [pltpu.VMEM((ROWS,), jnp.int32) for _ in range(NBUF)]
            + [pltpu.VMEM((ROWS, FEAT), jnp.float32) for _ in range(NBUF)]
            + [pltpu.VMEM((BLK, FEAT), jnp.float32) for _ in range(NBUF)]
            + [pltpu.SemaphoreType.DMA for _ in range(3 * NBUF)]
        ),
        compiler_params=pltpu.CompilerParams(needs_layout_passes=False),
    )
    return f(x, neigh_orders)
